# baseline (speedup 1.0000x reference)
import math

import jax
import jax.numpy as jnp
from jax import lax
from jax.experimental import pallas as pl
from jax.experimental.pallas import tpu as pltpu
from jax.experimental.pallas import tpu_sc as plsc

D_MODEL = 1024
SCALE = math.sqrt(D_MODEL)

_info = plsc.get_sparse_core_info()
_NC, _NS, _L = _info.num_cores, _info.num_subcores, _info.num_lanes
_NW = _NC * _NS

_CHUNK = 32
_NBUF = 3
_VECS_PER_ROW = D_MODEL // _L


def _emb_body(table_hbm, x_hbm, out_hbm, idx_v, bufs, gsems, wsems):
    wid = lax.axis_index("s") * _NC + lax.axis_index("c")
    seq = x_hbm.shape[1]
    bpw = x_hbm.shape[0] * seq // _NW
    wper = seq // bpw
    bb = wid // wper
    off = (wid % wper) * bpw
    pltpu.sync_copy(x_hbm.at[bb, pl.ds(off, bpw)], idx_v)
    nchunks = bpw // _CHUNK

    def gather_start(k, b):
        pltpu.async_copy(table_hbm.at[idx_v.at[pl.ds(k * _CHUNK, _CHUNK)]],
                         bufs[b], gsems[b])

    def gather_wait(k, b):
        pltpu.make_async_copy(table_hbm.at[idx_v.at[pl.ds(k * _CHUNK, _CHUNK)]],
                              bufs[b], gsems[b]).wait()

    def scatter_start_half(k, b, h):
        hc = _CHUNK // 2
        pltpu.async_copy(bufs[b].at[pl.ds(h * hc, hc)],
                         out_hbm.at[bb, pl.ds(off + k * _CHUNK + h * hc, hc)],
                         wsems[b])

    def scatter_wait(k, b):
        pltpu.make_async_copy(bufs[b],
                              out_hbm.at[bb, pl.ds(off + k * _CHUNK, _CHUNK)],
                              wsems[b]).wait()

    def scale_half(b, h):
        hc = _CHUNK // 2

        @plsc.parallel_loop(h * hc, (h + 1) * hc, unroll=2)
        def _(r):
            for j in range(_VECS_PER_ROW):
                col = j * _L
                bufs[b][r, pl.ds(col, _L)] = bufs[b][r, pl.ds(col, _L)] * SCALE

    gather_start(0, 0)
    gather_start(1, 1)

    def body(ci, carry):
        for j in range(_NBUF):
            k = ci * _NBUF + j

            @pl.when(k < nchunks)
            def _():
                b = j
                bn = (j + 2) % _NBUF
                gather_wait(k, b)

                @pl.when(k >= 1)
                def _():
                    scatter_wait(k - 1, bn)

                @pl.when(k + 2 < nchunks)
                def _():
                    gather_start(k + 2, bn)

                scale_half(b, 0)
                scatter_start_half(k, b, 0)
                scale_half(b, 1)
                scatter_start_half(k, b, 1)
        return carry

    nit = (nchunks + _NBUF - 1) // _NBUF
    lax.fori_loop(0, nit, body, 0)
    scatter_wait(nchunks - 1, (nchunks - 1) % _NBUF)


def kernel(table, x):
    mesh = plsc.VectorSubcoreMesh(core_axis_name="c", subcore_axis_name="s")
    run = pl.kernel(
        lambda t, xx, o, idx_v, b0, b1, b2, g0, g1, g2, w0, w1, w2:
            _emb_body(t, xx, o, idx_v, (b0, b1, b2), (g0, g1, g2),
                      (w0, w1, w2)),
        out_type=jax.ShapeDtypeStruct(x.shape + (D_MODEL,), jnp.float32),
        mesh=mesh,
        scratch_types=(
            [pltpu.VMEM((x.size // _NW,), jnp.int32)]
            + [pltpu.VMEM((_CHUNK, D_MODEL), jnp.float32)] * _NBUF
            + [pltpu.SemaphoreType.DMA] * (2 * _NBUF)
        ),
    )
    return run(table, x.astype(jnp.int32))


# speedup vs baseline: 1.3637x; 1.3637x over previous
import math

import jax
import jax.numpy as jnp
from jax import lax
from jax.experimental import pallas as pl
from jax.experimental.pallas import tpu as pltpu
from jax.experimental.pallas import tpu_sc as plsc

D_MODEL = 1024
SCALE = math.sqrt(D_MODEL)

_info = plsc.get_sparse_core_info()
_NC, _NS, _L = _info.num_cores, _info.num_subcores, _info.num_lanes
_NW = _NC * _NS

_CHUNK = 32
_NBUF = 3
_VECS_PER_ROW = D_MODEL // _L


def _emb_body(table_hbm, x_hbm, out_hbm, idx_v, bufs, gsems, wsems):
    wid = lax.axis_index("s") * _NC + lax.axis_index("c")
    seq = x_hbm.shape[1]
    bpw = x_hbm.shape[0] * seq // _NW
    wper = seq // bpw
    bb = wid // wper
    off = (wid % wper) * bpw
    pltpu.sync_copy(x_hbm.at[bb, pl.ds(off, bpw)], idx_v)
    nchunks = bpw // _CHUNK

    def gather_start(k, b):
        pltpu.async_copy(table_hbm.at[idx_v.at[pl.ds(k * _CHUNK, _CHUNK)]],
                         bufs[b], gsems[b])

    def gather_wait(k, b):
        pltpu.make_async_copy(table_hbm.at[idx_v.at[pl.ds(k * _CHUNK, _CHUNK)]],
                              bufs[b], gsems[b]).wait()

    def scatter_start(k, b):
        pltpu.async_copy(bufs[b],
                         out_hbm.at[bb, pl.ds(off + k * _CHUNK, _CHUNK)],
                         wsems[b])

    def scatter_wait(k, b):
        pltpu.make_async_copy(bufs[b],
                              out_hbm.at[bb, pl.ds(off + k * _CHUNK, _CHUNK)],
                              wsems[b]).wait()

    def scale(b):
        @plsc.parallel_loop(0, _CHUNK, unroll=2)
        def _(r):
            for j in range(_VECS_PER_ROW):
                col = j * _L
                bufs[b][r, pl.ds(col, _L)] = bufs[b][r, pl.ds(col, _L)] * SCALE

    gather_start(0, 0)
    gather_start(1, 1)

    def body(ci, carry):
        for j in range(_NBUF):
            k = ci * _NBUF + j

            @pl.when(k < nchunks)
            def _():
                b = j
                bn = (j + 2) % _NBUF
                gather_wait(k, b)

                @pl.when(k >= 1)
                def _():
                    scatter_wait(k - 1, bn)

                @pl.when(k + 2 < nchunks)
                def _():
                    gather_start(k + 2, bn)

                scale(b)
                scatter_start(k, b)
        return carry

    nit = (nchunks + _NBUF - 1) // _NBUF
    lax.fori_loop(0, nit, body, 0)
    scatter_wait(nchunks - 1, (nchunks - 1) % _NBUF)


def kernel(table, x):
    mesh = plsc.VectorSubcoreMesh(core_axis_name="c", subcore_axis_name="s")
    run = pl.kernel(
        lambda t, xx, o, idx_v, b0, b1, b2, g0, g1, g2, w0, w1, w2:
            _emb_body(t, xx, o, idx_v, (b0, b1, b2), (g0, g1, g2),
                      (w0, w1, w2)),
        out_type=jax.ShapeDtypeStruct(x.shape + (D_MODEL,), jnp.float32),
        mesh=mesh,
        scratch_types=(
            [pltpu.VMEM((x.size // _NW,), jnp.int32)]
            + [pltpu.VMEM((_CHUNK, D_MODEL), jnp.float32)] * _NBUF
            + [pltpu.SemaphoreType.DMA] * (2 * _NBUF)
        ),
    )
    return run(table, x.astype(jnp.int32))


# speedup vs baseline: 1.4708x; 1.0786x over previous
import math

import jax
import jax.numpy as jnp
from jax import lax
from jax.experimental import pallas as pl
from jax.experimental.pallas import tpu as pltpu
from jax.experimental.pallas import tpu_sc as plsc

D_MODEL = 1024
SCALE = math.sqrt(D_MODEL)

_info = plsc.get_sparse_core_info()
_NC, _NS, _L = _info.num_cores, _info.num_subcores, _info.num_lanes
_NW = _NC * _NS

_CHUNK = 40
_TAIL = 32
_VECS_PER_ROW = D_MODEL // _L


def _emb_body(table_hbm, x_hbm, out_hbm, idx_v, bufs, gsems, wsems):
    wid = lax.axis_index("s") * _NC + lax.axis_index("c")
    seq = x_hbm.shape[1]
    bpw = x_hbm.shape[0] * seq // _NW
    wper = seq // bpw
    bb = wid // wper
    off = (wid % wper) * bpw
    pltpu.sync_copy(x_hbm.at[bb, pl.ds(off, bpw)], idx_v)
    nmain = (bpw - _TAIL) // _CHUNK

    def gather_start(k, b):
        pltpu.async_copy(table_hbm.at[idx_v.at[pl.ds(k * _CHUNK, _CHUNK)]],
                         bufs[b], gsems[b])

    def gather_wait(k, b):
        pltpu.make_async_copy(table_hbm.at[idx_v.at[pl.ds(k * _CHUNK, _CHUNK)]],
                              bufs[b], gsems[b]).wait()

    def scatter_start(k, b):
        pltpu.async_copy(bufs[b],
                         out_hbm.at[bb, pl.ds(off + k * _CHUNK, _CHUNK)],
                         wsems[b])

    def scatter_wait(k, b):
        pltpu.make_async_copy(bufs[b],
                              out_hbm.at[bb, pl.ds(off + k * _CHUNK, _CHUNK)],
                              wsems[b]).wait()

    def scale(b, rows):
        @plsc.parallel_loop(0, rows, unroll=2)
        def _(r):
            for j in range(_VECS_PER_ROW):
                col = j * _L
                bufs[b][r, pl.ds(col, _L)] = bufs[b][r, pl.ds(col, _L)] * SCALE

    def tail_gather_start(b):
        pltpu.async_copy(
            table_hbm.at[idx_v.at[pl.ds(nmain * _CHUNK, _TAIL)]],
            bufs[b].at[pl.ds(0, _TAIL)], gsems[b])

    def tail_gather_wait(b):
        pltpu.make_async_copy(
            table_hbm.at[idx_v.at[pl.ds(nmain * _CHUNK, _TAIL)]],
            bufs[b].at[pl.ds(0, _TAIL)], gsems[b]).wait()

    def tail_scatter_start(b):
        pltpu.async_copy(bufs[b].at[pl.ds(0, _TAIL)],
                         out_hbm.at[bb, pl.ds(off + nmain * _CHUNK, _TAIL)],
                         wsems[b])

    def tail_scatter_wait(b):
        pltpu.make_async_copy(
            bufs[b].at[pl.ds(0, _TAIL)],
            out_hbm.at[bb, pl.ds(off + nmain * _CHUNK, _TAIL)],
            wsems[b]).wait()

    gather_start(0, 0)

    def body(ci, carry):
        k0 = 2 * ci
        k1 = k0 + 1
        gather_wait(k0, 0)

        @pl.when(ci >= 1)
        def _():
            scatter_wait(k0 - 1, 1)

        gather_start(k1, 1)
        scale(0, _CHUNK)
        scatter_start(k0, 0)

        gather_wait(k1, 1)
        scatter_wait(k0, 0)

        @pl.when(k0 + 2 < nmain)
        def _():
            gather_start(k0 + 2, 0)

        @pl.when(k0 + 2 == nmain)
        def _():
            tail_gather_start(0)

        scale(1, _CHUNK)
        scatter_start(k1, 1)
        return carry

    lax.fori_loop(0, nmain // 2, body, 0)
    tail_gather_wait(0)
    scale(0, _TAIL)
    tail_scatter_start(0)
    scatter_wait(nmain - 1, 1)
    tail_scatter_wait(0)


def kernel(table, x):
    mesh = plsc.VectorSubcoreMesh(core_axis_name="c", subcore_axis_name="s")
    run = pl.kernel(
        lambda t, xx, o, idx_v, b0, b1, g0, g1, w0, w1:
            _emb_body(t, xx, o, idx_v, (b0, b1), (g0, g1), (w0, w1)),
        out_type=jax.ShapeDtypeStruct(x.shape + (D_MODEL,), jnp.float32),
        mesh=mesh,
        scratch_types=(
            [pltpu.VMEM((x.size // _NW,), jnp.int32)]
            + [pltpu.VMEM((_CHUNK, D_MODEL), jnp.float32)] * 2
            + [pltpu.SemaphoreType.DMA] * 4
        ),
    )
    return run(table, x.astype(jnp.int32))


# speedup vs baseline: 1.5201x; 1.0335x over previous
import math

import jax
import jax.numpy as jnp
from jax import lax
from jax.experimental import pallas as pl
from jax.experimental.pallas import tpu as pltpu
from jax.experimental.pallas import tpu_sc as plsc

D_MODEL = 1024
SCALE = math.sqrt(D_MODEL)

_info = plsc.get_sparse_core_info()
_NC, _NS, _L = _info.num_cores, _info.num_subcores, _info.num_lanes
_NW = _NC * _NS

_CHUNK = 48
_TAIL = 32
_VECS_PER_ROW = D_MODEL // _L


def _emb_body(table_hbm, x_hbm, out_hbm, idx_v, bufs, gsems, wsems):
    wid = lax.axis_index("s") * _NC + lax.axis_index("c")
    seq = x_hbm.shape[1]
    bpw = x_hbm.shape[0] * seq // _NW
    wper = seq // bpw
    bb = wid // wper
    off = (wid % wper) * bpw
    pltpu.sync_copy(x_hbm.at[bb, pl.ds(off, bpw)], idx_v)
    nmain = (bpw - _TAIL) // _CHUNK

    def gather_start(k, b):
        pltpu.async_copy(table_hbm.at[idx_v.at[pl.ds(k * _CHUNK, _CHUNK)]],
                         bufs[b], gsems[b])

    def gather_wait(k, b):
        pltpu.make_async_copy(table_hbm.at[idx_v.at[pl.ds(k * _CHUNK, _CHUNK)]],
                              bufs[b], gsems[b]).wait()

    def scatter_start(k, b):
        pltpu.async_copy(bufs[b],
                         out_hbm.at[bb, pl.ds(off + k * _CHUNK, _CHUNK)],
                         wsems[b])

    def scatter_wait(k, b):
        pltpu.make_async_copy(bufs[b],
                              out_hbm.at[bb, pl.ds(off + k * _CHUNK, _CHUNK)],
                              wsems[b]).wait()

    def scale(b, rows):
        @plsc.parallel_loop(0, rows, unroll=2)
        def _(r):
            for j in range(_VECS_PER_ROW):
                col = j * _L
                bufs[b][r, pl.ds(col, _L)] = bufs[b][r, pl.ds(col, _L)] * SCALE

    def tail_gather_start(b):
        pltpu.async_copy(
            table_hbm.at[idx_v.at[pl.ds(nmain * _CHUNK, _TAIL)]],
            bufs[b].at[pl.ds(0, _TAIL)], gsems[b])

    def tail_gather_wait(b):
        pltpu.make_async_copy(
            table_hbm.at[idx_v.at[pl.ds(nmain * _CHUNK, _TAIL)]],
            bufs[b].at[pl.ds(0, _TAIL)], gsems[b]).wait()

    def tail_scatter_start(b):
        pltpu.async_copy(bufs[b].at[pl.ds(0, _TAIL)],
                         out_hbm.at[bb, pl.ds(off + nmain * _CHUNK, _TAIL)],
                         wsems[b])

    def tail_scatter_wait(b):
        pltpu.make_async_copy(
            bufs[b].at[pl.ds(0, _TAIL)],
            out_hbm.at[bb, pl.ds(off + nmain * _CHUNK, _TAIL)],
            wsems[b]).wait()

    gather_start(0, 0)

    def body(ci, carry):
        k0 = 2 * ci
        k1 = k0 + 1
        gather_wait(k0, 0)

        @pl.when(ci >= 1)
        def _():
            scatter_wait(k0 - 1, 1)

        gather_start(k1, 1)
        scale(0, _CHUNK)
        scatter_start(k0, 0)

        gather_wait(k1, 1)
        scatter_wait(k0, 0)

        @pl.when(k0 + 2 < nmain)
        def _():
            gather_start(k0 + 2, 0)

        @pl.when(k0 + 2 == nmain)
        def _():
            tail_gather_start(0)

        scale(1, _CHUNK)
        scatter_start(k1, 1)
        return carry

    lax.fori_loop(0, nmain // 2, body, 0)
    tail_gather_wait(0)
    scale(0, _TAIL)
    tail_scatter_start(0)
    scatter_wait(nmain - 1, 1)
    tail_scatter_wait(0)


def kernel(table, x):
    mesh = plsc.VectorSubcoreMesh(core_axis_name="c", subcore_axis_name="s")
    run = pl.kernel(
        lambda t, xx, o, idx_v, b0, b1, g0, g1, w0, w1:
            _emb_body(t, xx, o, idx_v, (b0, b1), (g0, g1), (w0, w1)),
        out_type=jax.ShapeDtypeStruct(x.shape + (D_MODEL,), jnp.float32),
        mesh=mesh,
        scratch_types=(
            [pltpu.VMEM((x.size // _NW,), jnp.int32)]
            + [pltpu.VMEM((_CHUNK, D_MODEL), jnp.float32)] * 2
            + [pltpu.SemaphoreType.DMA] * 4
        ),
    )
    return run(table, x.astype(jnp.int32))
